# Initial kernel scaffold; baseline (speedup 1.0000x reference)
import jax
import jax.numpy as jnp
from jax.experimental import pallas as pl


def kernel(bias_matrix, group_assignment):
    raise NotImplementedError("write your pallas kernel here")


# speedup vs baseline: 19.7231x; 19.7231x over previous
import functools

import jax
import jax.numpy as jnp
from jax import lax
from jax.experimental import pallas as pl
from jax.experimental.pallas import tpu as pltpu
from jax.experimental.pallas import tpu_sc as plsc

_SEQ = 256
_NG = 5
_NH = 32
_LANES = 16
_SUBCORES = 16


@jax.jit
def _fg_bias_sc(bias_flat, g):
    mesh = plsc.VectorSubcoreMesh(core_axis_name="c", subcore_axis_name="s")

    @functools.partial(
        pl.kernel,
        mesh=mesh,
        out_type=jax.ShapeDtypeStruct((_NH * _SEQ, _SEQ), jnp.float32),
        compiler_params=pltpu.CompilerParams(
            use_tc_tiling_on_sc=False, needs_layout_passes=False
        ),
        scratch_types=[
            pltpu.VMEM((_NH * _NG * _NG,), jnp.float32),
            pltpu.VMEM((_SEQ,), jnp.int32),
            pltpu.VMEM((8, _SEQ), jnp.float32),
            pltpu.VMEM((2, 128), jnp.int32),
            pltpu.VMEM((_SEQ, _SEQ), jnp.float32),
            pltpu.VMEM_SHARED((_SUBCORES * 8, _SEQ), jnp.float32),
            pltpu.SemaphoreType.DMA,
            pltpu.SemaphoreType.DMA,
        ],
    )
    def k(b_hbm, g_hbm, out_hbm, b_v, g_v, tmp_v, idx_v, out_v, tbl_sh, sem0, sem1):
        c = lax.axis_index("c")
        s = lax.axis_index("s")
        h = c * _SUBCORES + s
        pltpu.sync_copy(g_hbm, g_v)
        pltpu.sync_copy(b_hbm, b_v)

        base = h * (_NG * _NG)
        for a in range(_NG):
            for ch in range(_SEQ // _LANES):
                gj = g_v[pl.ds(ch * _LANES, _LANES)]
                tmp_v[a, pl.ds(ch * _LANES, _LANES)] = plsc.load_gather(
                    b_v, [gj + (base + a * _NG)]
                )

        pltpu.sync_copy(tmp_v, tbl_sh.at[pl.ds(s * 8, 8)])

        for ch in range(_SEQ // _LANES):
            gj = g_v[pl.ds(ch * _LANES, _LANES)]
            r, off = divmod(ch * _LANES, 128)
            idx_v[r, pl.ds(off, _LANES)] = gj + s * 8

        plsc.subcore_barrier()

        cp0 = pltpu.async_copy(tbl_sh.at[idx_v.at[0]], out_v.at[pl.ds(0, 128)], sem0)
        cp1 = pltpu.async_copy(tbl_sh.at[idx_v.at[1]], out_v.at[pl.ds(128, 128)], sem1)
        cp0.wait()
        cp1.wait()

        pltpu.sync_copy(out_v, out_hbm.at[pl.ds(h * _SEQ, _SEQ)])

    return k(bias_flat, g)


def kernel(bias_matrix, group_assignment):
    out = _fg_bias_sc(bias_matrix.reshape(-1), group_assignment)
    return out.reshape(_NH, _SEQ, _SEQ)


# speedup vs baseline: 20.2118x; 1.0248x over previous
import functools

import jax
import jax.numpy as jnp
from jax import lax
from jax.experimental import pallas as pl
from jax.experimental.pallas import tpu as pltpu
from jax.experimental.pallas import tpu_sc as plsc

_SEQ = 256
_NG = 5
_NH = 32
_LANES = 16
_SUBCORES = 16


@jax.jit
def _fg_bias_sc(bias_flat, g):
    mesh = plsc.VectorSubcoreMesh(core_axis_name="c", subcore_axis_name="s")

    @functools.partial(
        pl.kernel,
        mesh=mesh,
        out_type=jax.ShapeDtypeStruct((_NH * _SEQ, _SEQ), jnp.float32),
        compiler_params=pltpu.CompilerParams(
            use_tc_tiling_on_sc=False, needs_layout_passes=False
        ),
        scratch_types=[
            pltpu.VMEM((_NH * _NG * _NG,), jnp.float32),
            pltpu.VMEM((_SEQ,), jnp.int32),
            pltpu.VMEM((8, _SEQ), jnp.float32),
            pltpu.VMEM((4, 64), jnp.int32),
            pltpu.VMEM((_SEQ, _SEQ), jnp.float32),
            pltpu.VMEM_SHARED((_SUBCORES * 8, _SEQ), jnp.float32),
            [pltpu.SemaphoreType.DMA] * 4,
            [pltpu.SemaphoreType.DMA] * 4,
        ],
    )
    def k(b_hbm, g_hbm, out_hbm, b_v, g_v, tmp_v, idx_v, out_v, tbl_sh, gsem, wsem):
        c = lax.axis_index("c")
        s = lax.axis_index("s")
        h = c * _SUBCORES + s
        pltpu.sync_copy(g_hbm, g_v)
        pltpu.sync_copy(b_hbm, b_v)

        base = h * (_NG * _NG)
        for a in range(_NG):
            for ch in range(_SEQ // _LANES):
                gj = g_v[pl.ds(ch * _LANES, _LANES)]
                tmp_v[a, pl.ds(ch * _LANES, _LANES)] = plsc.load_gather(
                    b_v, [gj + (base + a * _NG)]
                )

        pltpu.sync_copy(tmp_v, tbl_sh.at[pl.ds(s * 8, 8)])

        for ch in range(_SEQ // _LANES):
            gj = g_v[pl.ds(ch * _LANES, _LANES)]
            r, off = divmod(ch * _LANES, 64)
            idx_v[r, pl.ds(off, _LANES)] = gj + s * 8

        plsc.subcore_barrier()

        ob = out_hbm.at[pl.ds(h * _SEQ, _SEQ)]
        gcp = [
            pltpu.async_copy(
                tbl_sh.at[idx_v.at[t]], out_v.at[pl.ds(t * 64, 64)], gsem[t]
            )
            for t in range(4)
        ]
        wcp = []
        for t in range(4):
            gcp[t].wait()
            wcp.append(
                pltpu.async_copy(
                    out_v.at[pl.ds(t * 64, 64)], ob.at[pl.ds(t * 64, 64)], wsem[t]
                )
            )
        for c in wcp:
            c.wait()

    return k(bias_flat, g)


def kernel(bias_matrix, group_assignment):
    out = _fg_bias_sc(bias_matrix.reshape(-1), group_assignment)
    return out.reshape(_NH, _SEQ, _SEQ)
